# baseline (speedup 1.0000x reference)
import functools

import jax
import jax.numpy as jnp
from jax import lax
from jax.experimental import pallas as pl
from jax.experimental.pallas import tpu as pltpu
from jax.experimental.pallas import tpu_sc as plsc

NUM_CLASSES = 1000
DIM = 128
BATCH = 16384

_info = plsc.get_sparse_core_info()
_NC, _NS = _info.num_cores, _info.num_subcores
_NW = _NC * _NS
_B_PER_W = BATCH // _NW
_NCHUNK = 8
_CHUNK = _B_PER_W // _NCHUNK
_RING = 4


@functools.partial(
    pl.kernel,
    mesh=plsc.VectorSubcoreMesh(core_axis_name="c", subcore_axis_name="s"),
    out_type=jax.ShapeDtypeStruct((BATCH, DIM), jnp.float32),
    scratch_types=[
        pltpu.VMEM((_B_PER_W,), jnp.int32),
        pltpu.VMEM((_RING, _CHUNK, DIM), jnp.float32),
        pltpu.VMEM_SHARED((NUM_CLASSES + 1, DIM), jnp.float32),
        pltpu.VMEM_SHARED((_NS, 2, _CHUNK, DIM), jnp.float32),
        pltpu.SemaphoreType.DMA,
        pltpu.SemaphoreType.DMA,
        pltpu.SemaphoreType.DMA,
        pltpu.SemaphoreType.DMA,
    ],
)
def _gather_kernel(labels_hbm, table_hbm, out_hbm, idx_v, rows_v, tab_s,
                   stage_s, gsem, ssem, xsem, dsem):
    sid = lax.axis_index("s")
    wid = sid * _NC + lax.axis_index("c")
    base = wid * _B_PER_W

    @pl.when(sid == 0)
    def _():
        pltpu.sync_copy(table_hbm, tab_s)

    pltpu.sync_copy(labels_hbm.at[pl.ds(base, _B_PER_W)], idx_v)
    plsc.subcore_barrier()

    gathers = [None] * _NCHUNK
    stores = [None] * _NCHUNK
    xcopies = [None] * _NCHUNK
    dmas = [None] * _NCHUNK
    waited = set()

    def _wait(kind, lst, c):
        if lst[c] is not None and (kind, c) not in waited:
            lst[c].wait()
            waited.add((kind, c))

    def start_consumer(c):
        gathers[c].wait()
        if c % 2 == 0:
            stores[c] = pltpu.async_copy(
                rows_v.at[c % _RING],
                out_hbm.at[pl.ds(base + c * _CHUNK, _CHUNK)],
                ssem,
            )
        else:
            if c >= 4:
                _wait("d", dmas, c - 4)
            xcopies[c] = pltpu.async_copy(
                rows_v.at[c % _RING],
                stage_s.at[sid, (c // 2) % 2],
                xsem,
            )

    def start_dma(c):
        _wait("x", xcopies, c)
        dmas[c] = pltpu.async_copy(
            stage_s.at[sid, (c // 2) % 2],
            out_hbm.at[pl.ds(base + c * _CHUNK, _CHUNK)],
            dsem,
        )

    for c in range(_NCHUNK):
        if c >= _RING:
            p = c - _RING
            if p % 2 == 0:
                _wait("s", stores, p)
            else:
                _wait("x", xcopies, p)
        gathers[c] = pltpu.async_copy(
            tab_s.at[idx_v.at[pl.ds(c * _CHUNK, _CHUNK)]],
            rows_v.at[c % _RING],
            gsem,
        )
        if c >= 1:
            start_consumer(c - 1)
        if c >= 2 and (c - 2) % 2 == 1:
            start_dma(c - 2)
    start_consumer(_NCHUNK - 1)
    for c in range(_NCHUNK):
        if c % 2 == 1 and dmas[c] is None:
            start_dma(c)
    for c in range(_NCHUNK):
        if c % 2 == 0:
            _wait("s", stores, c)
        else:
            _wait("d", dmas, c)


def kernel(labels, table):
    return _gather_kernel(labels.astype(jnp.int32), table)


# speedup vs baseline: 1.0573x; 1.0573x over previous
import functools

import jax
import jax.numpy as jnp
from jax import lax
from jax.experimental import pallas as pl
from jax.experimental.pallas import tpu as pltpu
from jax.experimental.pallas import tpu_sc as plsc

NUM_CLASSES = 1000
DIM = 128
BATCH = 16384

_info = plsc.get_sparse_core_info()
_NC, _NS = _info.num_cores, _info.num_subcores
_NW = _NC * _NS
_B_PER_W = BATCH // _NW

_SIZES = (32, 96, 128, 128, 128)
_OFFS = (0, 32, 128, 256, 384)
_NCHUNK = len(_SIZES)
_RING = 3
_BUF = max(_SIZES)
_TAB_SPLIT = 64


@functools.partial(
    pl.kernel,
    mesh=plsc.VectorSubcoreMesh(core_axis_name="c", subcore_axis_name="s"),
    out_type=jax.ShapeDtypeStruct((BATCH, DIM), jnp.float32),
    scratch_types=[
        pltpu.VMEM((_B_PER_W,), jnp.int32),
        pltpu.VMEM((_RING, _BUF, DIM), jnp.float32),
        pltpu.VMEM_SHARED((NUM_CLASSES + 1, DIM), jnp.float32),
        pltpu.SemaphoreType.DMA,
        pltpu.SemaphoreType.DMA,
    ],
)
def _gather_kernel(labels_hbm, table_hbm, out_hbm, idx_v, rows_v, tab_s,
                   gsem, ssem):
    sid = lax.axis_index("s")
    wid = sid * _NC + lax.axis_index("c")
    base = wid * _B_PER_W

    _tail = (NUM_CLASSES + 1) - (_NS - 1) * _TAB_SPLIT

    @pl.when(sid < _NS - 1)
    def _():
        toff = sid * _TAB_SPLIT
        pltpu.sync_copy(
            table_hbm.at[pl.ds(toff, _TAB_SPLIT)],
            tab_s.at[pl.ds(toff, _TAB_SPLIT)],
        )

    @pl.when(sid == _NS - 1)
    def _():
        toff = (_NS - 1) * _TAB_SPLIT
        pltpu.sync_copy(
            table_hbm.at[pl.ds(toff, _tail)], tab_s.at[pl.ds(toff, _tail)]
        )
    pltpu.sync_copy(labels_hbm.at[pl.ds(base, _B_PER_W)], idx_v)
    plsc.subcore_barrier()

    gathers = [None] * _NCHUNK
    stores = [None] * _NCHUNK

    def start_store(c):
        gathers[c].wait()
        stores[c] = pltpu.async_copy(
            rows_v.at[c % _RING].at[pl.ds(0, _SIZES[c])],
            out_hbm.at[pl.ds(base + _OFFS[c], _SIZES[c])],
            ssem,
        )

    for c in range(_NCHUNK):
        if c >= _RING:
            stores[c - _RING].wait()
        gathers[c] = pltpu.async_copy(
            tab_s.at[idx_v.at[pl.ds(_OFFS[c], _SIZES[c])]],
            rows_v.at[c % _RING].at[pl.ds(0, _SIZES[c])],
            gsem,
        )
        if c >= 1:
            start_store(c - 1)
    start_store(_NCHUNK - 1)
    for c in range(_NCHUNK - _RING, _NCHUNK):
        stores[c].wait()


def kernel(labels, table):
    return _gather_kernel(labels.astype(jnp.int32), table)


# speedup vs baseline: 1.0640x; 1.0063x over previous
import functools

import jax
import jax.numpy as jnp
from jax import lax
from jax.experimental import pallas as pl
from jax.experimental.pallas import tpu as pltpu
from jax.experimental.pallas import tpu_sc as plsc

NUM_CLASSES = 1000
DIM = 128
BATCH = 16384

_info = plsc.get_sparse_core_info()
_NC, _NS = _info.num_cores, _info.num_subcores
_NW = _NC * _NS
_B_PER_W = BATCH // _NW
_NCHUNK = 4
_CHUNK = _B_PER_W // _NCHUNK
_RING = 2


@functools.partial(
    pl.kernel,
    mesh=plsc.VectorSubcoreMesh(core_axis_name="c", subcore_axis_name="s"),
    out_type=jax.ShapeDtypeStruct((BATCH, DIM), jnp.float32),
    scratch_types=[
        pltpu.VMEM((_B_PER_W,), jnp.int32),
        pltpu.VMEM((_RING, _CHUNK, DIM), jnp.float32),
        pltpu.VMEM_SHARED((NUM_CLASSES + 1, DIM), jnp.float32),
        pltpu.SemaphoreType.DMA,
        pltpu.SemaphoreType.DMA,
    ],
)
def _gather_kernel(labels_hbm, table_hbm, out_hbm, idx_v, rows_v, tab_s, gsem, ssem):
    sid = lax.axis_index("s")
    wid = sid * _NC + lax.axis_index("c")
    base = wid * _B_PER_W
    @pl.when(sid == 0)
    def _():
        pltpu.sync_copy(table_hbm, tab_s)
    pltpu.sync_copy(labels_hbm.at[pl.ds(base, _B_PER_W)], idx_v)
    plsc.subcore_barrier()
    gathers = [None] * _NCHUNK
    stores = [None] * _NCHUNK
    for c in range(_NCHUNK):
        if c >= _RING:
            stores[c - _RING].wait()
        gathers[c] = pltpu.async_copy(
            tab_s.at[idx_v.at[pl.ds(c * _CHUNK, _CHUNK)]],
            rows_v.at[c % _RING],
            gsem,
        )
        if c >= 1:
            gathers[c - 1].wait()
            stores[c - 1] = pltpu.async_copy(
                rows_v.at[(c - 1) % _RING],
                out_hbm.at[pl.ds(base + (c - 1) * _CHUNK, _CHUNK)],
                ssem,
            )
    gathers[_NCHUNK - 1].wait()
    stores[_NCHUNK - 1] = pltpu.async_copy(
        rows_v.at[(_NCHUNK - 1) % _RING],
        out_hbm.at[pl.ds(base + (_NCHUNK - 1) * _CHUNK, _CHUNK)],
        ssem,
    )
    stores[_NCHUNK - 2].wait()
    stores[_NCHUNK - 1].wait()


def kernel(labels, table):
    return _gather_kernel(labels.astype(jnp.int32), table)
